# Initial kernel scaffold; baseline (speedup 1.0000x reference)
import jax
import jax.numpy as jnp
from jax.experimental import pallas as pl


def kernel(x, token_table, pos_table):
    raise NotImplementedError("write your pallas kernel here")


# speedup vs baseline: 3.3503x; 3.3503x over previous
import functools

import jax
import jax.numpy as jnp
from jax import lax
from jax.experimental import pallas as pl
from jax.experimental.pallas import tpu as pltpu
from jax.experimental.pallas import tpu_sc as plsc

_MAXLEN = 200
_D = 64
_BATCH = 4096
_N = _BATCH * _MAXLEN
_NC = 2
_NS = 16
_NW = _NC * _NS
_BPW = _N // _NW
_STEP = 400
_NSTEP = _BPW // _STEP
_LANES = 16
_VPR = _D // _LANES

_mesh = plsc.VectorSubcoreMesh(core_axis_name="c", subcore_axis_name="s")


@functools.partial(
    pl.kernel,
    mesh=_mesh,
    compiler_params=pltpu.CompilerParams(use_tc_tiling_on_sc=False),
    out_type=jax.ShapeDtypeStruct((_N, _D), jnp.float32),
    scratch_types=[
        pltpu.VMEM((_STEP,), jnp.int32),
        pltpu.VMEM((_STEP, _D), jnp.float32),
        pltpu.VMEM((_STEP, _D), jnp.float32),
        pltpu.SemaphoreType.DMA,
    ],
)
def _emb_kernel(x_hbm, tok_hbm, pos_hbm, out_hbm, idx_v, rows_v, pos_v, sem):
    wid = lax.axis_index("s") * _NC + lax.axis_index("c")
    base = wid * _BPW
    pltpu.sync_copy(pos_hbm, pos_v.at[pl.ds(0, _MAXLEN)])
    pltpu.sync_copy(pos_hbm, pos_v.at[pl.ds(_MAXLEN, _MAXLEN)])

    def step_body(ci, carry):
        off = base + ci * _STEP
        pltpu.sync_copy(x_hbm.at[pl.ds(off, _STEP)], idx_v)
        copies = []
        for j in range(0, _STEP, 128):
            n = min(128, _STEP - j)
            copies.append(
                pltpu.async_copy(
                    tok_hbm.at[idx_v.at[pl.ds(j, n)]],
                    rows_v.at[pl.ds(j, n)],
                    sem,
                )
            )
        for cp in copies:
            cp.wait()

        def add_body(r, c2):
            for v in range(_VPR):
                sl = pl.ds(v * _LANES, _LANES)
                rows_v[r, sl] = rows_v[r, sl] + pos_v[r, sl]
            return c2

        lax.fori_loop(0, _STEP, add_body, 0)
        pltpu.sync_copy(rows_v, out_hbm.at[pl.ds(off, _STEP)])
        return carry

    lax.fori_loop(0, _NSTEP, step_body, 0)


def kernel(x, token_table, pos_table):
    xf = x.reshape(-1).astype(jnp.int32)
    out = _emb_kernel(xf, token_table, pos_table)
    return out.reshape(_BATCH, _MAXLEN, _D)


# speedup vs baseline: 4.1297x; 1.2326x over previous
import functools

import jax
import jax.numpy as jnp
from jax import lax
from jax.experimental import pallas as pl
from jax.experimental.pallas import tpu as pltpu
from jax.experimental.pallas import tpu_sc as plsc

_MAXLEN = 200
_D = 64
_BATCH = 4096
_N = _BATCH * _MAXLEN
_NC = 2
_NS = 16
_NW = _NC * _NS
_BPW = _N // _NW
_STEP = _MAXLEN
_NSTEP = _BPW // _STEP
_NBUF = 4
_G1 = 128
_G2 = _STEP - _G1
_LANES = 16
_VPR = _D // _LANES

_mesh = plsc.VectorSubcoreMesh(core_axis_name="c", subcore_axis_name="s")


@functools.partial(
    pl.kernel,
    mesh=_mesh,
    compiler_params=pltpu.CompilerParams(use_tc_tiling_on_sc=False),
    out_type=jax.ShapeDtypeStruct((_N, _D), jnp.float32),
    scratch_types=[
        pltpu.VMEM((_BPW,), jnp.int32),
        pltpu.VMEM((_NBUF, _STEP, _D), jnp.float32),
        pltpu.VMEM((_STEP, _D), jnp.float32),
    ]
    + [pltpu.SemaphoreType.DMA] * (2 * _NBUF),
)
def _emb_kernel(x_hbm, tok_hbm, pos_hbm, out_hbm, idx_v, rows_v, pos_v,
                sg0, sg1, sg2, sg3, so0, so1, so2, so3):
    sem_g = (sg0, sg1, sg2, sg3)
    sem_o = (so0, so1, so2, so3)
    wid = lax.axis_index("s") * _NC + lax.axis_index("c")
    base = wid * _BPW
    pltpu.sync_copy(x_hbm.at[pl.ds(base, _BPW)], idx_v)
    pltpu.sync_copy(pos_hbm, pos_v)

    def issue_gather(s, slot):
        lo = s * _STEP
        pltpu.async_copy(tok_hbm.at[idx_v.at[pl.ds(lo, _G1)]],
                         rows_v.at[slot, pl.ds(0, _G1)], sem_g[slot])
        pltpu.async_copy(tok_hbm.at[idx_v.at[pl.ds(lo + _G1, _G2)]],
                         rows_v.at[slot, pl.ds(_G1, _G2)], sem_g[slot])

    def wait_gather(slot):
        pltpu.make_async_copy(out_hbm.at[pl.ds(0, _STEP)],
                              rows_v.at[slot], sem_g[slot]).wait()

    def add_pos(slot):
        def add_body(i, c):
            for u in range(2):
                r = i * 2 + u
                for v in range(_VPR):
                    sl = pl.ds(v * _LANES, _LANES)
                    rows_v[slot, r, sl] = rows_v[slot, r, sl] + pos_v[r, sl]
            return c

        lax.fori_loop(0, _STEP // 2, add_body, 0)

    def issue_out(s, slot):
        off = base + s * _STEP
        pltpu.async_copy(rows_v.at[slot], out_hbm.at[pl.ds(off, _STEP)],
                         sem_o[slot])

    def wait_out(slot):
        pltpu.make_async_copy(rows_v.at[slot], out_hbm.at[pl.ds(0, _STEP)],
                              sem_o[slot]).wait()

    issue_gather(0, 0)
    issue_gather(1, 1)

    for b in range(_NBUF):
        wait_gather(b)
        add_pos(b)
        issue_out(b, b)
        slot2 = (b + 2) % _NBUF
        if b >= 2:
            wait_out(slot2)
        issue_gather(b + 2, slot2)

    def outer(g, c):
        for b in range(_NBUF):
            s = g * _NBUF + b
            wait_gather(b)
            add_pos(b)
            issue_out(s, b)
            slot2 = (b + 2) % _NBUF
            wait_out(slot2)
            issue_gather(s + 2, slot2)
        return c

    lax.fori_loop(1, _NSTEP // _NBUF - 1, outer, 0)

    for b in range(_NBUF):
        s = _NSTEP - _NBUF + b
        wait_gather(b)
        add_pos(b)
        issue_out(s, b)
        if b < 2:
            slot2 = (b + 2) % _NBUF
            wait_out(slot2)
            issue_gather(s + 2, slot2)
    for b in range(_NBUF):
        wait_out(b)


def kernel(x, token_table, pos_table):
    xf = x.reshape(-1).astype(jnp.int32)
    out = _emb_kernel(xf, token_table, pos_table)
    return out.reshape(_BATCH, _MAXLEN, _D)


# speedup vs baseline: 4.1339x; 1.0010x over previous
import functools

import jax
import jax.numpy as jnp
from jax import lax
from jax.experimental import pallas as pl
from jax.experimental.pallas import tpu as pltpu
from jax.experimental.pallas import tpu_sc as plsc

_MAXLEN = 200
_D = 64
_BATCH = 4096
_NC = 2
_NS = 16
_NW = _NC * _NS
_BPW = _BATCH // _NW
_NBUF = 4
_G1 = 128
_G2 = _MAXLEN - _G1
_LANES = 16
_VPR = _D // _LANES

_mesh = plsc.VectorSubcoreMesh(core_axis_name="c", subcore_axis_name="s")


@functools.partial(
    pl.kernel,
    mesh=_mesh,
    compiler_params=pltpu.CompilerParams(use_tc_tiling_on_sc=False),
    out_type=jax.ShapeDtypeStruct((_BATCH, _MAXLEN, _D), jnp.float32),
    scratch_types=[
        pltpu.VMEM((_BPW, _MAXLEN), jnp.int32),
        pltpu.VMEM((_NBUF, _MAXLEN, _D), jnp.float32),
        pltpu.VMEM((_MAXLEN, _D), jnp.float32),
    ]
    + [pltpu.SemaphoreType.DMA] * (2 * _NBUF),
)
def _emb_kernel(x_hbm, tok_hbm, pos_hbm, out_hbm, idx_v, rows_v, pos_v,
                sg0, sg1, sg2, sg3, so0, so1, so2, so3):
    sem_g = (sg0, sg1, sg2, sg3)
    sem_o = (so0, so1, so2, so3)
    wid = lax.axis_index("s") * _NC + lax.axis_index("c")
    bbase = wid * _BPW
    pltpu.sync_copy(x_hbm.at[pl.ds(bbase, _BPW)], idx_v)
    pltpu.sync_copy(pos_hbm, pos_v)

    def issue_gather(s, slot):
        pltpu.async_copy(tok_hbm.at[idx_v.at[s, pl.ds(0, _G1)]],
                         rows_v.at[slot, pl.ds(0, _G1)], sem_g[slot])
        pltpu.async_copy(tok_hbm.at[idx_v.at[s, pl.ds(_G1, _G2)]],
                         rows_v.at[slot, pl.ds(_G1, _G2)], sem_g[slot])

    def wait_gather(slot):
        pltpu.make_async_copy(out_hbm.at[0], rows_v.at[slot],
                              sem_g[slot]).wait()

    def add_pos(slot):
        def add_body(i, c):
            for u in range(2):
                r = i * 2 + u
                for v in range(_VPR):
                    sl = pl.ds(v * _LANES, _LANES)
                    rows_v[slot, r, sl] = rows_v[slot, r, sl] + pos_v[r, sl]
            return c

        lax.fori_loop(0, _MAXLEN // 2, add_body, 0)

    def issue_out(s, slot):
        pltpu.async_copy(rows_v.at[slot], out_hbm.at[bbase + s], sem_o[slot])

    def wait_out(slot):
        pltpu.make_async_copy(rows_v.at[slot], out_hbm.at[0],
                              sem_o[slot]).wait()

    issue_gather(0, 0)
    issue_gather(1, 1)

    for b in range(_NBUF):
        wait_gather(b)
        add_pos(b)
        issue_out(b, b)
        slot2 = (b + 2) % _NBUF
        if b >= 2:
            wait_out(slot2)
        issue_gather(b + 2, slot2)

    def outer(g, c):
        for b in range(_NBUF):
            s = g * _NBUF + b
            wait_gather(b)
            add_pos(b)
            issue_out(s, b)
            slot2 = (b + 2) % _NBUF
            wait_out(slot2)
            issue_gather(s + 2, slot2)
        return c

    lax.fori_loop(1, _BPW // _NBUF - 1, outer, 0)

    for b in range(_NBUF):
        s = _BPW - _NBUF + b
        wait_gather(b)
        add_pos(b)
        issue_out(s, b)
        if b < 2:
            slot2 = (b + 2) % _NBUF
            wait_out(slot2)
            issue_gather(s + 2, slot2)
    for b in range(_NBUF):
        wait_out(b)


def kernel(x, token_table, pos_table):
    return _emb_kernel(x.astype(jnp.int32), token_table, pos_table)
